# baseline (speedup 1.0000x reference)
import functools

import jax
import jax.numpy as jnp
from jax import lax
from jax.experimental import pallas as pl
from jax.experimental.pallas import tpu as pltpu
from jax.experimental.pallas import tpu_sc as plsc

NC = 2
NS = 16
L = 16
NW = NC * NS

VOCAB = 1000000
NUM_IDX = 16384
EMBED = 64
PER_W = NUM_IDX // NW
CHUNK = 128
NCHUNK = PER_W // CHUNK


def _sc_counts(idx3, zeros_hbm):
    mesh = plsc.VectorSubcoreMesh(core_axis_name="c", subcore_axis_name="s")

    @functools.partial(
        pl.kernel,
        out_type=(
            jax.ShapeDtypeStruct((VOCAB,), jnp.float32),
            jax.ShapeDtypeStruct((VOCAB,), jnp.float32),
        ),
        mesh=mesh,
        scratch_types=[
            pltpu.VMEM((NCHUNK, CHUNK), jnp.int32),
            pltpu.VMEM((CHUNK,), jnp.float32),
            pltpu.VMEM_SHARED((VOCAB,), jnp.float32),
        ],
    )
    def k(idx_hbm, z_hbm, out0, out1, idx_v, ones_v, counts_s):
        cid = lax.axis_index("c")
        sid = lax.axis_index("s")
        wid = sid * NC + cid
        pltpu.sync_copy(idx_hbm.at[wid], idx_v)
        for i in range(CHUNK // L):
            ones_v[pl.ds(i * L, L)] = jnp.ones((L,), jnp.float32)

        @pl.when(sid == 0)
        def _():
            pltpu.sync_copy(z_hbm, counts_s)

        plsc.subcore_barrier()
        for j in range(NCHUNK):
            pltpu.sync_copy(ones_v, counts_s.at[idx_v.at[j]], add=True)
        plsc.subcore_barrier()

        @pl.when(jnp.logical_and(sid == 0, cid == 0))
        def _():
            pltpu.sync_copy(counts_s, out0)

        @pl.when(jnp.logical_and(sid == 0, cid == 1))
        def _():
            pltpu.sync_copy(counts_s, out1)

    return k(idx3, zeros_hbm)


BL = 1024
STEPS = -(-VOCAB // BL)
NPAD = STEPS * BL


def _tc_bag_matvec(tableT, counts_pad):
    def body(n_ref, t_ref, o_ref):
        s = pl.program_id(0)

        @pl.when(s == 0)
        def _():
            o_ref[...] = jnp.zeros_like(o_ref)

        lane = s * BL + lax.broadcasted_iota(jnp.int32, (1, BL), 1)
        t = jnp.where(lane < VOCAB, t_ref[...], 0.0)
        n2 = n_ref[pl.ds(s * BL, BL)].reshape(8, 128)
        acc = o_ref[...]
        for q in range(BL // 128):
            acc = acc + lax.dot_general(
                n2[q : q + 1, :], t[:, q * 128 : (q + 1) * 128],
                (((1,), (1,)), ((), ())),
                precision=lax.Precision.HIGHEST,
                preferred_element_type=jnp.float32,
            )
        o_ref[...] = acc

    return pl.pallas_call(
        body,
        grid=(STEPS,),
        in_specs=[
            pl.BlockSpec((NPAD,), lambda s: (0,)),
            pl.BlockSpec((EMBED, BL), lambda s: (0, s)),
        ],
        out_specs=pl.BlockSpec((1, EMBED), lambda s: (0, 0)),
        out_shape=jax.ShapeDtypeStruct((1, EMBED), jnp.float32),
    )(counts_pad, tableT)


def _tc_mlp(summed, W1, b1, W2, b2):
    def body(p_ref, w1_ref, b1_ref, w2_ref, b2_ref, o_ref):
        s = p_ref[...]
        h = lax.dot_general(
            s, w1_ref[...], (((1,), (1,)), ((), ())),
            precision=lax.Precision.DEFAULT,
            preferred_element_type=jnp.float32,
        )
        h = jnp.maximum(h + b1_ref[...], 0.0)
        h_r = h.astype(jnp.bfloat16).astype(jnp.float32)
        w2_r = w2_ref[...].astype(jnp.bfloat16).astype(jnp.float32)
        o_ref[0, 0] = jnp.sum(h_r * w2_r) + b2_ref[0, 0]

    return pl.pallas_call(
        body,
        out_shape=jax.ShapeDtypeStruct((1, 1), jnp.float32),
        in_specs=[
            pl.BlockSpec(memory_space=pltpu.VMEM),
            pl.BlockSpec(memory_space=pltpu.VMEM),
            pl.BlockSpec(memory_space=pltpu.VMEM),
            pl.BlockSpec(memory_space=pltpu.VMEM),
            pl.BlockSpec(memory_space=pltpu.SMEM),
        ],
        out_specs=pl.BlockSpec(memory_space=pltpu.SMEM),
    )(summed, W1, b1.reshape(1, -1), W2, b2.reshape(1, 1))


def kernel(indices, table, W1, b1, W2, b2):
    idx3 = indices.astype(jnp.int32).reshape(NW, NCHUNK, CHUNK)
    zeros = jnp.zeros((VOCAB,), jnp.float32)
    c0, c1 = _sc_counts(idx3, zeros)
    counts_pad = jnp.pad(c0 + c1, (0, NPAD - VOCAB))
    summed = _tc_bag_matvec(table.T, counts_pad)
    out = _tc_mlp(summed, W1, b1, W2, b2)
    return out.reshape(1)


# speedup vs baseline: 3.4363x; 3.4363x over previous
import functools

import jax
import jax.numpy as jnp
from jax import lax
from jax.experimental import pallas as pl
from jax.experimental.pallas import tpu as pltpu
from jax.experimental.pallas import tpu_sc as plsc

NC = 2
NS = 16
L = 16
NW = NC * NS

VOCAB = 1000000
NUM_IDX = 16384
EMBED = 64
PER_W = NUM_IDX // NW
CHUNK = 128
NCHUNK = PER_W // CHUNK


def _sc_counts(idx3, zeros_hbm):
    mesh = plsc.VectorSubcoreMesh(core_axis_name="c", subcore_axis_name="s")

    @functools.partial(
        pl.kernel,
        out_type=(
            jax.ShapeDtypeStruct((VOCAB,), jnp.float32),
            jax.ShapeDtypeStruct((VOCAB,), jnp.float32),
        ),
        mesh=mesh,
        scratch_types=[
            pltpu.VMEM((NCHUNK, CHUNK), jnp.int32),
            pltpu.VMEM((CHUNK,), jnp.float32),
            pltpu.VMEM_SHARED((VOCAB,), jnp.float32),
        ],
    )
    def k(idx_hbm, z_hbm, out0, out1, idx_v, ones_v, counts_s):
        cid = lax.axis_index("c")
        sid = lax.axis_index("s")
        wid = sid * NC + cid
        pltpu.sync_copy(idx_hbm.at[wid], idx_v)
        for i in range(CHUNK // L):
            ones_v[pl.ds(i * L, L)] = jnp.ones((L,), jnp.float32)

        @pl.when(sid == 0)
        def _():
            pltpu.sync_copy(z_hbm, counts_s)

        plsc.subcore_barrier()
        for j in range(NCHUNK):
            pltpu.sync_copy(ones_v, counts_s.at[idx_v.at[j]], add=True)
        plsc.subcore_barrier()

        @pl.when(jnp.logical_and(sid == 0, cid == 0))
        def _():
            pltpu.sync_copy(counts_s, out0)

        @pl.when(jnp.logical_and(sid == 0, cid == 1))
        def _():
            pltpu.sync_copy(counts_s, out1)

    return k(idx3, zeros_hbm)


BL = 4096
STEPS = -(-VOCAB // BL)
NPAD = STEPS * BL


def _tc_bag_matvec(tableT, counts_pad):
    def body(n_ref, t_ref, o_ref):
        s = pl.program_id(0)

        @pl.when(s == 0)
        def _():
            o_ref[...] = jnp.zeros_like(o_ref)

        lane = s * BL + lax.broadcasted_iota(jnp.int32, (1, BL), 1)
        t = jnp.where(lane < VOCAB, t_ref[...], 0.0)
        n = n_ref[pl.ds(s * BL, BL)]
        o_ref[...] += t * n[None, :]

    return pl.pallas_call(
        body,
        grid=(STEPS,),
        in_specs=[
            pl.BlockSpec((NPAD,), lambda s: (0,)),
            pl.BlockSpec((EMBED, BL), lambda s: (0, s)),
        ],
        out_specs=pl.BlockSpec((EMBED, BL), lambda s: (0, 0)),
        out_shape=jax.ShapeDtypeStruct((EMBED, BL), jnp.float32),
    )(counts_pad, tableT)


def _tc_mlp(acc, W1, b1, W2, b2):
    def body(p_ref, w1_ref, b1_ref, w2_ref, b2_ref, o_ref):
        s = jnp.sum(p_ref[...], axis=1, keepdims=True)
        h = lax.dot_general(
            w1_ref[...], s, (((1,), (0,)), ((), ())),
            precision=lax.Precision.DEFAULT,
            preferred_element_type=jnp.float32,
        )
        h = jnp.maximum(h + b1_ref[...], 0.0)
        h_r = h.astype(jnp.bfloat16).astype(jnp.float32)
        w2_r = w2_ref[...].astype(jnp.bfloat16).astype(jnp.float32)
        o_ref[0, 0] = jnp.sum(h_r * w2_r) + b2_ref[0, 0]

    return pl.pallas_call(
        body,
        out_shape=jax.ShapeDtypeStruct((1, 1), jnp.float32),
        in_specs=[
            pl.BlockSpec(memory_space=pltpu.VMEM),
            pl.BlockSpec(memory_space=pltpu.VMEM),
            pl.BlockSpec(memory_space=pltpu.VMEM),
            pl.BlockSpec(memory_space=pltpu.VMEM),
            pl.BlockSpec(memory_space=pltpu.SMEM),
        ],
        out_specs=pl.BlockSpec(memory_space=pltpu.SMEM),
    )(acc, W1, b1.reshape(-1, 1), W2.reshape(-1, 1), b2.reshape(1, 1))


def kernel(indices, table, W1, b1, W2, b2):
    idx3 = indices.astype(jnp.int32).reshape(NW, NCHUNK, CHUNK)
    zeros = jnp.zeros((VOCAB,), jnp.float32)
    c0, c1 = _sc_counts(idx3, zeros)
    counts_pad = jnp.pad(c0 + c1, (0, NPAD - VOCAB))
    acc = _tc_bag_matvec(table.T, counts_pad)
    out = _tc_mlp(acc, W1, b1, W2, b2)
    return out.reshape(1)


# speedup vs baseline: 4.5896x; 1.3356x over previous
import functools

import jax
import jax.numpy as jnp
from jax import lax
from jax.experimental import pallas as pl
from jax.experimental.pallas import tpu as pltpu
from jax.experimental.pallas import tpu_sc as plsc

NC = 2
NS = 16
L = 16
NW = NC * NS

VOCAB = 1000000
NUM_IDX = 16384
EMBED = 64
PER_W = NUM_IDX // NW
CHUNK = 128
NCHUNK = PER_W // CHUNK


def _sc_counts(idx3, zeros_hbm):
    mesh = plsc.VectorSubcoreMesh(core_axis_name="c", subcore_axis_name="s")

    @functools.partial(
        pl.kernel,
        out_type=(
            jax.ShapeDtypeStruct((VOCAB,), jnp.float32),
            jax.ShapeDtypeStruct((VOCAB,), jnp.float32),
        ),
        mesh=mesh,
        scratch_types=[
            pltpu.VMEM((NCHUNK, CHUNK), jnp.int32),
            pltpu.VMEM((CHUNK,), jnp.float32),
            pltpu.VMEM_SHARED((VOCAB,), jnp.float32),
        ],
    )
    def k(idx_hbm, z_hbm, out0, out1, idx_v, ones_v, counts_s):
        cid = lax.axis_index("c")
        sid = lax.axis_index("s")
        wid = sid * NC + cid
        pltpu.sync_copy(idx_hbm.at[wid], idx_v)
        for i in range(CHUNK // L):
            ones_v[pl.ds(i * L, L)] = jnp.ones((L,), jnp.float32)

        @pl.when(sid == 0)
        def _():
            pltpu.sync_copy(z_hbm, counts_s)

        plsc.subcore_barrier()
        for j in range(NCHUNK):
            pltpu.sync_copy(ones_v, counts_s.at[idx_v.at[j]], add=True)
        plsc.subcore_barrier()

        @pl.when(jnp.logical_and(sid == 0, cid == 0))
        def _():
            pltpu.sync_copy(counts_s, out0)

        @pl.when(jnp.logical_and(sid == 0, cid == 1))
        def _():
            pltpu.sync_copy(counts_s, out1)

    return k(idx3, zeros_hbm)


BL = 8192
STEPS = -(-VOCAB // BL)
NPAD = STEPS * BL


def _tc_bag_matvec(tableT, counts_pad):
    def body(n_ref, t_ref, o_ref):
        s = pl.program_id(0)

        @pl.when(s == 0)
        def _():
            o_ref[...] = jnp.zeros_like(o_ref)

        lane = s * BL + lax.broadcasted_iota(jnp.int32, (1, BL), 1)
        t = jnp.where(lane < VOCAB, t_ref[...], 0.0)
        n = n_ref[pl.ds(s * BL, BL)]
        o_ref[...] += t * n[None, :]

    return pl.pallas_call(
        body,
        grid=(STEPS,),
        in_specs=[
            pl.BlockSpec((NPAD,), lambda s: (0,)),
            pl.BlockSpec((EMBED, BL), lambda s: (0, s)),
        ],
        out_specs=pl.BlockSpec((EMBED, BL), lambda s: (0, 0)),
        out_shape=jax.ShapeDtypeStruct((EMBED, BL), jnp.float32),
    )(counts_pad, tableT)


def _tc_mlp(acc, W1, b1, W2, b2):
    def body(p_ref, w1_ref, b1_ref, w2_ref, b2_ref, o_ref):
        s = jnp.sum(p_ref[...], axis=1, keepdims=True)
        h = lax.dot_general(
            w1_ref[...], s, (((1,), (0,)), ((), ())),
            precision=lax.Precision.DEFAULT,
            preferred_element_type=jnp.float32,
        )
        h = jnp.maximum(h + b1_ref[...], 0.0)
        h_r = h.astype(jnp.bfloat16).astype(jnp.float32)
        w2_r = w2_ref[...].astype(jnp.bfloat16).astype(jnp.float32)
        o_ref[0, 0] = jnp.sum(h_r * w2_r) + b2_ref[0, 0]

    return pl.pallas_call(
        body,
        out_shape=jax.ShapeDtypeStruct((1, 1), jnp.float32),
        in_specs=[
            pl.BlockSpec(memory_space=pltpu.VMEM),
            pl.BlockSpec(memory_space=pltpu.VMEM),
            pl.BlockSpec(memory_space=pltpu.VMEM),
            pl.BlockSpec(memory_space=pltpu.VMEM),
            pl.BlockSpec(memory_space=pltpu.SMEM),
        ],
        out_specs=pl.BlockSpec(memory_space=pltpu.SMEM),
    )(acc, W1, b1.reshape(-1, 1), W2.reshape(-1, 1), b2.reshape(1, 1))


def kernel(indices, table, W1, b1, W2, b2):
    idx3 = indices.astype(jnp.int32).reshape(NW, NCHUNK, CHUNK)
    zeros = jnp.zeros((VOCAB,), jnp.float32)
    c0, c1 = _sc_counts(idx3, zeros)
    counts_pad = jnp.pad(c0 + c1, (0, NPAD - VOCAB))
    acc = _tc_bag_matvec(table.T, counts_pad)
    out = _tc_mlp(acc, W1, b1, W2, b2)
    return out.reshape(1)


# speedup vs baseline: 5.5220x; 1.2032x over previous
import functools

import jax
import jax.numpy as jnp
from jax import lax
from jax.experimental import pallas as pl
from jax.experimental.pallas import tpu as pltpu
from jax.experimental.pallas import tpu_sc as plsc

NC = 2
NS = 16
L = 16
NW = NC * NS

VOCAB = 1000000
NUM_IDX = 16384
EMBED = 64
PER_W = NUM_IDX // NW
CHUNK = 128
NCHUNK = PER_W // CHUNK


def _sc_counts(idx3, zeros_hbm):
    mesh = plsc.VectorSubcoreMesh(core_axis_name="c", subcore_axis_name="s")

    @functools.partial(
        pl.kernel,
        out_type=(
            jax.ShapeDtypeStruct((VOCAB,), jnp.float32),
            jax.ShapeDtypeStruct((VOCAB,), jnp.float32),
        ),
        mesh=mesh,
        scratch_types=[
            pltpu.VMEM((NCHUNK, CHUNK), jnp.int32),
            pltpu.VMEM((CHUNK,), jnp.float32),
            pltpu.VMEM_SHARED((VOCAB,), jnp.float32),
        ],
    )
    def k(idx_hbm, z_hbm, out0, out1, idx_v, ones_v, counts_s):
        cid = lax.axis_index("c")
        sid = lax.axis_index("s")
        wid = sid * NC + cid
        pltpu.sync_copy(idx_hbm.at[wid], idx_v)
        for i in range(CHUNK // L):
            ones_v[pl.ds(i * L, L)] = jnp.ones((L,), jnp.float32)

        @pl.when(sid == 0)
        def _():
            pltpu.sync_copy(z_hbm, counts_s)

        plsc.subcore_barrier()
        for j in range(NCHUNK):
            pltpu.sync_copy(ones_v, counts_s.at[idx_v.at[j]], add=True)
        plsc.subcore_barrier()

        @pl.when(jnp.logical_and(sid == 0, cid == 0))
        def _():
            pltpu.sync_copy(counts_s, out0)

        @pl.when(jnp.logical_and(sid == 0, cid == 1))
        def _():
            pltpu.sync_copy(counts_s, out1)

    return k(idx3, zeros_hbm)


BL = 16384
STEPS = -(-VOCAB // BL)
NPAD = STEPS * BL


def _tc_bag_matvec(tableT, counts_pad):
    def body(n_ref, t_ref, o_ref):
        s = pl.program_id(0)

        @pl.when(s == 0)
        def _():
            o_ref[...] = jnp.zeros_like(o_ref)

        lane = s * BL + lax.broadcasted_iota(jnp.int32, (1, BL), 1)
        t = jnp.where(lane < VOCAB, t_ref[...], 0.0)
        n = n_ref[pl.ds(s * BL, BL)]
        o_ref[...] += t * n[None, :]

    return pl.pallas_call(
        body,
        grid=(STEPS,),
        in_specs=[
            pl.BlockSpec((NPAD,), lambda s: (0,)),
            pl.BlockSpec((EMBED, BL), lambda s: (0, s)),
        ],
        out_specs=pl.BlockSpec((EMBED, BL), lambda s: (0, 0)),
        out_shape=jax.ShapeDtypeStruct((EMBED, BL), jnp.float32),
    )(counts_pad, tableT)


def _tc_mlp(acc, W1, b1, W2, b2):
    def body(p_ref, w1_ref, b1_ref, w2_ref, b2_ref, o_ref):
        s = jnp.sum(p_ref[...], axis=1, keepdims=True)
        h = lax.dot_general(
            w1_ref[...], s, (((1,), (0,)), ((), ())),
            precision=lax.Precision.DEFAULT,
            preferred_element_type=jnp.float32,
        )
        h = jnp.maximum(h + b1_ref[...], 0.0)
        h_r = h.astype(jnp.bfloat16).astype(jnp.float32)
        w2_r = w2_ref[...].astype(jnp.bfloat16).astype(jnp.float32)
        o_ref[0, 0] = jnp.sum(h_r * w2_r) + b2_ref[0, 0]

    return pl.pallas_call(
        body,
        out_shape=jax.ShapeDtypeStruct((1, 1), jnp.float32),
        in_specs=[
            pl.BlockSpec(memory_space=pltpu.VMEM),
            pl.BlockSpec(memory_space=pltpu.VMEM),
            pl.BlockSpec(memory_space=pltpu.VMEM),
            pl.BlockSpec(memory_space=pltpu.VMEM),
            pl.BlockSpec(memory_space=pltpu.SMEM),
        ],
        out_specs=pl.BlockSpec(memory_space=pltpu.SMEM),
    )(acc, W1, b1.reshape(-1, 1), W2.reshape(-1, 1), b2.reshape(1, 1))


def kernel(indices, table, W1, b1, W2, b2):
    idx3 = indices.astype(jnp.int32).reshape(NW, NCHUNK, CHUNK)
    zeros = jnp.zeros((VOCAB,), jnp.float32)
    c0, c1 = _sc_counts(idx3, zeros)
    counts_pad = jnp.pad(c0 + c1, (0, NPAD - VOCAB))
    acc = _tc_bag_matvec(table.T, counts_pad)
    out = _tc_mlp(acc, W1, b1, W2, b2)
    return out.reshape(1)


# speedup vs baseline: 6.0559x; 1.0967x over previous
import functools

import jax
import jax.numpy as jnp
from jax import lax
from jax.experimental import pallas as pl
from jax.experimental.pallas import tpu as pltpu
from jax.experimental.pallas import tpu_sc as plsc

NC = 2
NS = 16
L = 16
NW = NC * NS

VOCAB = 1000000
NUM_IDX = 16384
EMBED = 64
PER_W = NUM_IDX // NW
CHUNK = 128
NCHUNK = PER_W // CHUNK


def _sc_counts(idx3, zeros_hbm):
    mesh = plsc.VectorSubcoreMesh(core_axis_name="c", subcore_axis_name="s")

    @functools.partial(
        pl.kernel,
        out_type=(
            jax.ShapeDtypeStruct((VOCAB,), jnp.float32),
            jax.ShapeDtypeStruct((VOCAB,), jnp.float32),
        ),
        mesh=mesh,
        scratch_types=[
            pltpu.VMEM((NCHUNK, CHUNK), jnp.int32),
            pltpu.VMEM((CHUNK,), jnp.float32),
            pltpu.VMEM_SHARED((VOCAB,), jnp.float32),
        ],
    )
    def k(idx_hbm, z_hbm, out0, out1, idx_v, ones_v, counts_s):
        cid = lax.axis_index("c")
        sid = lax.axis_index("s")
        wid = sid * NC + cid
        pltpu.sync_copy(idx_hbm.at[wid], idx_v)
        for i in range(CHUNK // L):
            ones_v[pl.ds(i * L, L)] = jnp.ones((L,), jnp.float32)

        @pl.when(sid == 0)
        def _():
            pltpu.sync_copy(z_hbm, counts_s)

        plsc.subcore_barrier()
        for j in range(NCHUNK):
            pltpu.sync_copy(ones_v, counts_s.at[idx_v.at[j]], add=True)
        plsc.subcore_barrier()

        @pl.when(jnp.logical_and(sid == 0, cid == 0))
        def _():
            pltpu.sync_copy(counts_s, out0)

        @pl.when(jnp.logical_and(sid == 0, cid == 1))
        def _():
            pltpu.sync_copy(counts_s, out1)

    return k(idx3, zeros_hbm)


BL = 32768
STEPS = -(-VOCAB // BL)
NPAD = STEPS * BL


def _tc_bag_matvec(tableT, counts_pad):
    def body(n_ref, t_ref, o_ref):
        s = pl.program_id(0)

        @pl.when(s == 0)
        def _():
            o_ref[...] = jnp.zeros_like(o_ref)

        lane = s * BL + lax.broadcasted_iota(jnp.int32, (1, BL), 1)
        t = jnp.where(lane < VOCAB, t_ref[...], 0.0)
        n = n_ref[pl.ds(s * BL, BL)]
        o_ref[...] += t * n[None, :]

    return pl.pallas_call(
        body,
        grid=(STEPS,),
        in_specs=[
            pl.BlockSpec((NPAD,), lambda s: (0,)),
            pl.BlockSpec((EMBED, BL), lambda s: (0, s)),
        ],
        out_specs=pl.BlockSpec((EMBED, BL), lambda s: (0, 0)),
        out_shape=jax.ShapeDtypeStruct((EMBED, BL), jnp.float32),
    )(counts_pad, tableT)


def _tc_mlp(acc, W1, b1, W2, b2):
    def body(p_ref, w1_ref, b1_ref, w2_ref, b2_ref, o_ref):
        s = jnp.sum(p_ref[...], axis=1, keepdims=True)
        h = lax.dot_general(
            w1_ref[...], s, (((1,), (0,)), ((), ())),
            precision=lax.Precision.DEFAULT,
            preferred_element_type=jnp.float32,
        )
        h = jnp.maximum(h + b1_ref[...], 0.0)
        h_r = h.astype(jnp.bfloat16).astype(jnp.float32)
        w2_r = w2_ref[...].astype(jnp.bfloat16).astype(jnp.float32)
        o_ref[0, 0] = jnp.sum(h_r * w2_r) + b2_ref[0, 0]

    return pl.pallas_call(
        body,
        out_shape=jax.ShapeDtypeStruct((1, 1), jnp.float32),
        in_specs=[
            pl.BlockSpec(memory_space=pltpu.VMEM),
            pl.BlockSpec(memory_space=pltpu.VMEM),
            pl.BlockSpec(memory_space=pltpu.VMEM),
            pl.BlockSpec(memory_space=pltpu.VMEM),
            pl.BlockSpec(memory_space=pltpu.SMEM),
        ],
        out_specs=pl.BlockSpec(memory_space=pltpu.SMEM),
    )(acc, W1, b1.reshape(-1, 1), W2.reshape(-1, 1), b2.reshape(1, 1))


def kernel(indices, table, W1, b1, W2, b2):
    idx3 = indices.astype(jnp.int32).reshape(NW, NCHUNK, CHUNK)
    zeros = jnp.zeros((VOCAB,), jnp.float32)
    c0, c1 = _sc_counts(idx3, zeros)
    counts_pad = jnp.pad(c0 + c1, (0, NPAD - VOCAB))
    acc = _tc_bag_matvec(table.T, counts_pad)
    out = _tc_mlp(acc, W1, b1, W2, b2)
    return out.reshape(1)
